# baseline (speedup 1.0000x reference)
import jax
import jax.numpy as jnp
from jax.experimental import pallas as pl
from jax.experimental.pallas import tpu as pltpu


def _grouping_body(q0_ref, c1_ref, g0_ref, g1_ref):
    C, R, W = g0_ref.shape
    g0_ref[...] = jnp.broadcast_to(q0_ref[...], (C, R, W))
    g1_ref[...] = jnp.broadcast_to(c1_ref[0][:, None, :], (C, R, W))


def kernel(cloud0, cloud1):
    B0, C, N0 = cloud0.shape
    B1, _, N1 = cloud1.shape
    M0, M1 = B0 * N0, B1 * N1
    pts0 = jnp.transpose(cloud0, (0, 2, 1)).reshape(M0, C)
    pts1 = jnp.transpose(cloud1, (0, 2, 1)).reshape(M1, C)
    q0 = jnp.transpose(cloud0, (1, 0, 2)).reshape(C, M0, 1)

    R = 128
    grid = (M0 // R,)
    g0t, g1t = pl.pallas_call(
        _grouping_body,
        grid=grid,
        in_specs=[
            pl.BlockSpec((C, R, 1), lambda r: (0, r, 0)),
            pl.BlockSpec((1, C, N1), lambda r: (r * R // N0, 0, 0)),
        ],
        out_specs=[
            pl.BlockSpec((C, R, N1), lambda r: (0, r, 0)),
            pl.BlockSpec((C, R, N1), lambda r: (0, r, 0)),
        ],
        out_shape=[
            jax.ShapeDtypeStruct((C, M0, N1), jnp.float32),
            jax.ShapeDtypeStruct((C, M0, N1), jnp.float32),
        ],
        compiler_params=pltpu.CompilerParams(
            dimension_semantics=("parallel",),
        ),
    )(q0, cloud1)
    return (
        pts0,
        pts1,
        jnp.transpose(g0t, (1, 2, 0)),
        jnp.transpose(g1t, (1, 2, 0)),
    )


# speedup vs baseline: 1.0063x; 1.0063x over previous
import jax
import jax.numpy as jnp
from jax.experimental import pallas as pl
from jax.experimental.pallas import tpu as pltpu


def _grouping_body(q0_ref, c1_ref, g0_ref, g1_ref):
    C, R, W = g0_ref.shape
    g0_ref[...] = jnp.broadcast_to(q0_ref[...], (C, R, W))
    g1_ref[...] = jnp.broadcast_to(c1_ref[0][:, None, :], (C, R, W))


def kernel(cloud0, cloud1):
    B0, C, N0 = cloud0.shape
    B1, _, N1 = cloud1.shape
    M0, M1 = B0 * N0, B1 * N1
    pts0 = jnp.transpose(cloud0, (0, 2, 1)).reshape(M0, C)
    pts1 = jnp.transpose(cloud1, (0, 2, 1)).reshape(M1, C)
    q0 = jnp.transpose(cloud0, (1, 0, 2)).reshape(C, M0, 1)

    R = 256
    grid = (M0 // R,)
    g0t, g1t = pl.pallas_call(
        _grouping_body,
        grid=grid,
        in_specs=[
            pl.BlockSpec((C, R, 1), lambda r: (0, r, 0)),
            pl.BlockSpec((1, C, N1), lambda r: (r * R // N0, 0, 0)),
        ],
        out_specs=[
            pl.BlockSpec((C, R, N1), lambda r: (0, r, 0)),
            pl.BlockSpec((C, R, N1), lambda r: (0, r, 0)),
        ],
        out_shape=[
            jax.ShapeDtypeStruct((C, M0, N1), jnp.float32),
            jax.ShapeDtypeStruct((C, M0, N1), jnp.float32),
        ],
        compiler_params=pltpu.CompilerParams(
            dimension_semantics=("parallel",),
        ),
    )(q0, cloud1)
    return (
        pts0,
        pts1,
        jnp.transpose(g0t, (1, 2, 0)),
        jnp.transpose(g1t, (1, 2, 0)),
    )
